# Initial kernel scaffold; baseline (speedup 1.0000x reference)
import jax
import jax.numpy as jnp
from jax.experimental import pallas as pl


def kernel(T, thermal_capacity, L, conductivity, A, time_step, edge_index):
    raise NotImplementedError("write your pallas kernel here")


# speedup vs baseline: 118.9977x; 118.9977x over previous
import functools

import jax
import jax.numpy as jnp
from jax import lax
from jax.experimental import pallas as pl
from jax.experimental.pallas import tpu as pltpu
from jax.experimental.pallas import tpu_sc as plsc

N = 100000
E = 6400000
NW = 32
EPW = E // NW
K = 2000
NCHUNK = EPW // K
LANES = 16
NPAD = 100096
SLICE = NPAD // 16
CBRT_MAGIC = 710235477


def _cbrt(x):
    i = lax.bitcast_convert_type(x, jnp.int32)
    y = lax.bitcast_convert_type(lax.div(i, jnp.int32(3)) + CBRT_MAGIC,
                                 jnp.float32)
    third = jnp.float32(1.0 / 3.0)
    for _ in range(3):
        y = (y + y + x / (y * y)) * third
    return y


def _sc_body(t_hbm, cp_hbm, src_hbm, dst_hbm, l_hbm, c_hbm, a_hbm, dt_hbm,
             out_hbm,
             src_v, dst_v, l_v, c_v, a_v, ts_v, td_v, cs_v, cd_v,
             val_v, nval_v, dt_v, z_v, accum, sem):
    c = lax.axis_index("c")
    s = lax.axis_index("s")
    wid = c * 16 + s

    def zinit(i, _):
        z_v[pl.ds(i * LANES, LANES)] = jnp.zeros((LANES,), jnp.float32)
        return 0
    lax.fori_loop(0, SLICE // LANES, zinit, 0)
    pltpu.sync_copy(z_v, accum.at[pl.ds(s * SLICE, SLICE)])
    pltpu.sync_copy(dt_hbm, dt_v)
    plsc.subcore_barrier()

    def chunk(t, _):
        base = wid * EPW + t * K
        esl = pl.ds(base, K)
        pltpu.sync_copy(src_hbm.at[esl], src_v)
        pltpu.sync_copy(dst_hbm.at[esl], dst_v)
        pltpu.sync_copy(l_hbm.at[esl], l_v)
        pltpu.sync_copy(c_hbm.at[esl], c_v)
        pltpu.sync_copy(a_hbm.at[esl], a_v)
        pltpu.async_copy(t_hbm.at[src_v], ts_v, sem).wait()
        pltpu.async_copy(t_hbm.at[dst_v], td_v, sem).wait()
        pltpu.async_copy(cp_hbm.at[src_v], cs_v, sem).wait()
        pltpu.async_copy(cp_hbm.at[dst_v], cd_v, sem).wait()

        def step(j, _):
            sl = pl.ds(j * LANES, LANES)
            ts = ts_v[sl]
            td = td_v[sl]
            delta = jnp.maximum(ts - td, jnp.float32(0.0))
            x = delta / l_v[sl] * c_v[sl]
            x = jnp.where(x > 0, x, jnp.float32(0.0))
            hf = _cbrt(x) * a_v[sl]
            ec = hf * dt_v[pl.ds(0, LANES)]
            cs = cs_v[sl]
            cd = cd_v[sl]
            ccp = cd * cs / (cd + cs)
            e = jnp.minimum(ec, delta * ccp)
            e = jnp.where(x > 0, e, jnp.float32(0.0))
            val_v[sl] = e
            nval_v[sl] = -e
            return 0
        lax.fori_loop(0, K // LANES, step, 0)

        pltpu.sync_copy(val_v, accum.at[dst_v], add=True)
        pltpu.sync_copy(nval_v, accum.at[src_v], add=True)
        return 0
    lax.fori_loop(0, NCHUNK, chunk, 0)

    plsc.subcore_barrier()
    pltpu.sync_copy(accum.at[pl.ds(s * SLICE, SLICE)], z_v)
    pltpu.sync_copy(z_v, out_hbm.at[pl.ds(c * NPAD + s * SLICE, SLICE)])


_sc_kernel = functools.partial(
    pl.kernel,
    out_type=jax.ShapeDtypeStruct((2 * NPAD,), jnp.float32),
    mesh=plsc.VectorSubcoreMesh(core_axis_name="c", subcore_axis_name="s"),
    scratch_types=[
        pltpu.VMEM((K,), jnp.int32),
        pltpu.VMEM((K,), jnp.int32),
        pltpu.VMEM((K,), jnp.float32),
        pltpu.VMEM((K,), jnp.float32),
        pltpu.VMEM((K,), jnp.float32),
        pltpu.VMEM((K,), jnp.float32),
        pltpu.VMEM((K,), jnp.float32),
        pltpu.VMEM((K,), jnp.float32),
        pltpu.VMEM((K,), jnp.float32),
        pltpu.VMEM((K,), jnp.float32),
        pltpu.VMEM((K,), jnp.float32),
        pltpu.VMEM((LANES,), jnp.float32),
        pltpu.VMEM((SLICE,), jnp.float32),
        pltpu.VMEM_SHARED((NPAD,), jnp.float32),
        pltpu.SemaphoreType.DMA,
    ],
)(_sc_body)


def _reduce_body(a_ref, o_ref):
    o_ref[...] = a_ref[0] + a_ref[1]


def _reduce2(a):
    a3 = a.reshape(2, NPAD // 128, 128)
    out = pl.pallas_call(
        _reduce_body,
        out_shape=jax.ShapeDtypeStruct((NPAD // 128, 128), jnp.float32),
    )(a3)
    return out.reshape(NPAD)[:N]


def kernel(T, thermal_capacity, L, conductivity, A, time_step, edge_index):
    dt16 = jnp.broadcast_to(time_step, (LANES,))
    partial = _sc_kernel(T, thermal_capacity, edge_index[0], edge_index[1],
                         L, conductivity, A, dt16)
    return _reduce2(partial)
